# baseline (speedup 1.0000x reference)
import functools

import jax
import jax.numpy as jnp
from jax.experimental import pallas as pl
from jax.experimental.pallas import tpu as pltpu

NEG_SLOPE = 0.2

_CFG = [
    (6,   64,  5, 1, 2),
    (64,  64,  4, 2, 1),
    (64,  128, 3, 1, 1),
    (128, 128, 4, 2, 1),
    (128, 256, 3, 1, 1),
    (256, 256, 4, 2, 1),
    (256, 512, 3, 1, 1),
    (512, 512, 4, 2, 1),
    (512, 512, 3, 1, 1),
    (512, 512, 4, 2, 1),
]


def _lrelu_store(acc, b_ref, o_ref, neg_slope):
    acc = acc + b_ref[...]
    acc = jnp.where(acc > 0, acc, neg_slope * acc)
    o_ref[...] = acc.astype(o_ref.dtype).reshape(o_ref.shape)


def _conv_s1_kernel(q_ref, w_ref, b_ref, o_ref, *, ksize, tr, w2, nb, c_in,
                    shared, neg_slope):
    m2 = nb * tr * w2
    r0 = pl.program_id(2) * tr

    def act(a):
        return jnp.where(a > 0, a, neg_slope * a)

    if shared:
        parts = [q_ref[:, pl.ds(r0 + kh, tr), pl.ds(j, w2), :]
                 for kh in range(ksize) for j in range(3)]
        patch = jnp.concatenate(parts, axis=-1).reshape(m2, w_ref.shape[0])
        acc = jnp.dot(patch, w_ref[...], preferred_element_type=jnp.float32)
        out = act(acc + b_ref[...])
    else:
        halves = []
        for off in (0, 1):
            parts = []
            for kh in range(ksize):
                for t in range(ksize):
                    idx = t + off + 1
                    j, half = idx // 2, idx % 2
                    parts.append(q_ref[:, pl.ds(r0 + kh, tr), pl.ds(j, w2),
                                       half * c_in:(half + 1) * c_in])
            patch = jnp.concatenate(parts, axis=-1).reshape(
                m2, ksize * ksize * c_in)
            a = jnp.dot(patch, w_ref[...].reshape(-1, w_ref.shape[-1]),
                        preferred_element_type=jnp.float32)
            halves.append(act(a + b_ref[...]))
        out = jnp.concatenate(halves, axis=-1)
    o_ref[...] = out.astype(o_ref.dtype).reshape(o_ref.shape)


def _conv_s2_kernel(x_ref, w_ref, b_ref, o_ref, *, tr, w_out, nb, c_in,
                    neg_slope):
    m = nb * tr * w_out
    kk = w_ref.shape[1]
    r0 = pl.program_id(2) * tr

    acc = jnp.zeros((m, w_ref.shape[-1]), jnp.float32)
    for kh in range(4):
        hq, hp = r0 + kh // 2, kh % 2
        if c_in >= 128:
            parts = []
            for kw in range(4):
                j = (kw + 1) // 2
                lo = c_in if kw % 2 == 0 else 0
                parts.append(x_ref[:, pl.ds(hq, tr), hp, pl.ds(j, w_out),
                                   lo:lo + c_in])
        else:
            parts = [x_ref[:, pl.ds(hq, tr), hp, pl.ds(j, w_out), :]
                     for j in range(3)]
        patch = jnp.concatenate(parts, axis=-1).reshape(m, kk)
        acc = acc + jnp.dot(patch, w_ref[kh],
                            preferred_element_type=jnp.float32)
    _lrelu_store(acc, b_ref, o_ref, neg_slope)


def _pick_tiles(n, h_out, w_out, c_t):
    max_rows = max(1, (1024 * 1024) // (w_out * c_t * 4))
    tr = 1
    for d in range(1, h_out + 1):
        if h_out % d == 0 and d <= max_rows:
            tr = d
    nb = 1
    if tr == h_out:
        while (nb * 2 <= n and n % (nb * 2) == 0
               and 2 * nb * h_out * w_out <= 512):
            nb *= 2
    return tr, nb


def _conv_s1(x, w, b, *, ksize, padding, c_in, c_out, neg_slope=NEG_SLOPE):
    n, h, w2, _ = x.shape
    shared = w.ndim == 2
    h_out = h

    qp = jnp.pad(x, ((0, 0), (padding, padding), (1, 1), (0, 0)))
    tr, nb = _pick_tiles(n, h_out, w2, 2 * c_out)
    n_rb = h_out // tr

    if shared:
        w_spec = pl.BlockSpec(w.shape, lambda ci, bi, mi: (0, 0))
    else:
        w_spec = pl.BlockSpec(w.shape, lambda ci, bi, mi: (0, 0, 0))

    kern = functools.partial(_conv_s1_kernel, ksize=ksize, tr=tr, w2=w2,
                             nb=nb, c_in=c_in, shared=shared,
                             neg_slope=neg_slope)
    out = pl.pallas_call(
        kern,
        out_shape=jax.ShapeDtypeStruct((n, h_out * w2, 2 * c_out), x.dtype),
        grid=(1, n // nb, n_rb),
        in_specs=[
            pl.BlockSpec((nb,) + qp.shape[1:], lambda ci, bi, mi: (bi, 0, 0, 0)),
            w_spec,
            pl.BlockSpec((1, b.shape[-1]), lambda ci, bi, mi: (0, 0)),
        ],
        out_specs=pl.BlockSpec((nb, tr * w2, 2 * c_out),
                               lambda ci, bi, mi: (bi, mi, 0)),
        compiler_params=pltpu.CompilerParams(
            dimension_semantics=("parallel", "parallel", "parallel"),
            vmem_limit_bytes=100 * 1024 * 1024),
    )(qp, w, b)
    return out.reshape(n, h_out, w2, 2 * c_out)


def _conv_s2(x, w, b, *, neg_slope=NEG_SLOPE):
    n, h, w2, c2 = x.shape
    c_in = c2 // 2
    c_out = w.shape[-1]
    h_out = h // 2
    w_out = w2

    xpp = jnp.pad(x, ((0, 0), (1, 3), (1, 1), (0, 0)))
    xpp = xpp.reshape(n, (h + 4) // 2, 2, w2 + 2, c2)
    c_t = min(c_out, 256)
    n_cb = c_out // c_t
    tr, nb = _pick_tiles(n, h_out, w_out, c_t)
    n_rb = h_out // tr

    kern = functools.partial(_conv_s2_kernel, tr=tr, w_out=w_out, nb=nb,
                             c_in=c_in, neg_slope=neg_slope)
    out = pl.pallas_call(
        kern,
        out_shape=jax.ShapeDtypeStruct((n, h_out * w_out, c_out), x.dtype),
        grid=(n_cb, n // nb, n_rb),
        in_specs=[
            pl.BlockSpec((nb,) + xpp.shape[1:],
                         lambda ci, bi, mi: (bi, 0, 0, 0, 0)),
            pl.BlockSpec((4, w.shape[1], c_t), lambda ci, bi, mi: (0, 0, ci)),
            pl.BlockSpec((1, c_t), lambda ci, bi, mi: (0, ci)),
        ],
        out_specs=pl.BlockSpec((nb, tr * w_out, c_t),
                               lambda ci, bi, mi: (bi, mi, ci)),
        compiler_params=pltpu.CompilerParams(
            dimension_semantics=("parallel", "parallel", "parallel"),
            vmem_limit_bytes=100 * 1024 * 1024),
    )(xpp, w, b)
    return out.reshape(n, h_out, w_out // 2, 2 * c_out)


def _head_kernel(x_ref, w1_ref, b1_ref, w2_ref, b2_ref, o_ref, *, neg_slope):
    h = jnp.dot(x_ref[...], w1_ref[...],
                preferred_element_type=jnp.float32) + b1_ref[...]
    h = jnp.where(h > 0, h, neg_slope * h).astype(jnp.bfloat16)
    y = jnp.dot(h, w2_ref[...], preferred_element_type=jnp.float32)
    o_ref[...] = y + b2_ref[...]


def _head(feat, w1, b1, w2, b2, *, neg_slope=NEG_SLOPE):
    n, f = feat.shape
    out = pl.pallas_call(
        functools.partial(_head_kernel, neg_slope=neg_slope),
        out_shape=jax.ShapeDtypeStruct((n, 1), jnp.float32),
        grid=(1,),
        in_specs=[
            pl.BlockSpec((n, f), lambda i: (0, 0)),
            pl.BlockSpec(w1.shape, lambda i: (0, 0)),
            pl.BlockSpec((1, w1.shape[1]), lambda i: (0, 0)),
            pl.BlockSpec(w2.shape, lambda i: (0, 0)),
            pl.BlockSpec((1, 1), lambda i: (0, 0)),
        ],
        out_specs=pl.BlockSpec((n, 1), lambda i: (0, 0)),
        compiler_params=pltpu.CompilerParams(
            dimension_semantics=("arbitrary",),
            vmem_limit_bytes=32 * 1024 * 1024),
    )(feat, w1, b1, w2, b2)
    return out.reshape(-1)


def kernel(inp, label, cw0, cb0, cw1, cb1, cw2, cb2, cw3, cb3, cw4, cb4,
           cw5, cb5, cw6, cb6, cw7, cb7, cw8, cb8, cw9, cb9, w1, b1, w2, b2):
    cws = [cw0, cw1, cw2, cw3, cw4, cw5, cw6, cw7, cw8, cw9]
    cbs = [cb0, cb1, cb2, cb3, cb4, cb5, cb6, cb7, cb8, cb9]

    x = jnp.concatenate([jnp.transpose(inp, (0, 2, 3, 1)),
                         jnp.transpose(label, (0, 2, 3, 1))], axis=-1)
    x = jnp.pad(x, ((0, 0), (0, 0), (0, 0), (0, 2))).astype(jnp.bfloat16)
    n, hh, ww, _ = x.shape
    x = x.reshape(n, hh, ww // 2, 16)

    for li, (c_in, c_out, k, s, p) in enumerate(_CFG):
        w, b = cws[li], cbs[li]
        if li == 0:
            c_in = 8
            w = jnp.pad(w, ((0, 0), (0, 0), (0, 2), (0, 0)))
        if s == 1:
            wr = w.reshape(k, k * c_in, c_out)
            if c_in < 128:
                ze = jnp.zeros((k, (2 - p) * c_in, c_out), w.dtype)
                zet = jnp.zeros((k, (6 - k - (2 - p)) * c_in, c_out), w.dtype)
                zo = jnp.zeros((k, (3 - p) * c_in, c_out), w.dtype)
                zot = jnp.zeros((k, (6 - k - (3 - p)) * c_in, c_out), w.dtype)
                we = jnp.concatenate([ze, wr, zet], axis=1)
                wo = jnp.concatenate([zo, wr, zot], axis=1)
                wk = jnp.concatenate([we, wo], axis=-1)
                wk = wk.reshape(k * 6 * c_in, 2 * c_out).astype(jnp.bfloat16)
                bb = jnp.concatenate([b, b]).reshape(1, 2 * c_out)
            else:
                wk = wr.astype(jnp.bfloat16)
                bb = b.reshape(1, c_out)
            x = _conv_s1(x, wk, bb, ksize=k, padding=p, c_in=c_in,
                         c_out=c_out)
        else:
            wk = w.reshape(k, k * c_in, c_out)
            if c_in < 128:
                wk = jnp.pad(wk, ((0, 0), (c_in, c_in), (0, 0)))
            wk = wk.astype(jnp.bfloat16)
            x = _conv_s2(x, wk, b.reshape(1, c_out))

    n = x.shape[0]
    feat = x.reshape(n, -1)
    w1p = (w1.reshape(512, 4, 4, 64).transpose(1, 2, 0, 3)
           .reshape(4 * 4 * 512, 64)).astype(jnp.bfloat16)
    return _head(feat, w1p, b1.reshape(1, -1), w2.astype(jnp.bfloat16),
                 b2.reshape(1, 1))


# speedup vs baseline: 1.0090x; 1.0090x over previous
import functools

import jax
import jax.numpy as jnp
from jax.experimental import pallas as pl
from jax.experimental.pallas import tpu as pltpu

NEG_SLOPE = 0.2

_CFG = [
    (6,   64,  5, 1, 2),
    (64,  64,  4, 2, 1),
    (64,  128, 3, 1, 1),
    (128, 128, 4, 2, 1),
    (128, 256, 3, 1, 1),
    (256, 256, 4, 2, 1),
    (256, 512, 3, 1, 1),
    (512, 512, 4, 2, 1),
    (512, 512, 3, 1, 1),
    (512, 512, 4, 2, 1),
]


def _lrelu_store(acc, b_ref, o_ref, neg_slope):
    acc = acc + b_ref[...]
    acc = jnp.where(acc > 0, acc, neg_slope * acc)
    o_ref[...] = acc.astype(o_ref.dtype).reshape(o_ref.shape)


def _conv_s1_kernel(q_ref, w_ref, b_ref, o_ref, *, ksize, tr, w2, nb, c_in,
                    shared, neg_slope):
    m2 = nb * tr * w2
    r0 = pl.program_id(2) * tr

    def act(a):
        return jnp.where(a > 0, a, neg_slope * a)

    if shared:
        parts = [q_ref[:, pl.ds(r0 + kh, tr), pl.ds(j, w2), :]
                 for kh in range(ksize) for j in range(3)]
        patch = jnp.concatenate(parts, axis=-1).reshape(m2, w_ref.shape[0])
        acc = jnp.dot(patch, w_ref[...], preferred_element_type=jnp.float32)
        out = act(acc + b_ref[...])
    else:
        halves = []
        for off in (0, 1):
            c_out = w_ref.shape[-1]
            a = jnp.zeros((m2, c_out), jnp.float32)
            for kh in range(ksize):
                pieces = []
                for t in range(ksize):
                    idx = t + off + 1
                    j, half = idx // 2, idx % 2
                    pieces.append((t, q_ref[:, pl.ds(r0 + kh, tr),
                                            pl.ds(j, w2),
                                            half * c_in:(half + 1) * c_in]))
                if c_in >= 256:
                    for t, piece in pieces:
                        a = a + jnp.dot(
                            piece.reshape(m2, c_in),
                            w_ref[kh, t * c_in:(t + 1) * c_in, :],
                            preferred_element_type=jnp.float32)
                else:
                    patch = jnp.concatenate(
                        [p for _, p in pieces],
                        axis=-1).reshape(m2, ksize * c_in)
                    a = a + jnp.dot(patch, w_ref[kh],
                                    preferred_element_type=jnp.float32)
            halves.append(act(a + b_ref[...]))
        out = jnp.concatenate(halves, axis=-1)
    o_ref[...] = out.astype(o_ref.dtype).reshape(o_ref.shape)


def _conv_s2_kernel(x_ref, w_ref, b_ref, o_ref, *, tr, w_out, nb, c_in,
                    neg_slope):
    m = nb * tr * w_out
    kk = w_ref.shape[1]
    r0 = pl.program_id(2) * tr

    acc = jnp.zeros((m, w_ref.shape[-1]), jnp.float32)
    for kh in range(4):
        hq, hp = r0 + kh // 2, kh % 2
        if c_in >= 256:
            for kw in range(4):
                j = (kw + 1) // 2
                lo = c_in if kw % 2 == 0 else 0
                piece = x_ref[:, pl.ds(hq, tr), hp, pl.ds(j, w_out),
                              lo:lo + c_in]
                acc = acc + jnp.dot(piece.reshape(m, c_in),
                                    w_ref[kh, kw * c_in:(kw + 1) * c_in, :],
                                    preferred_element_type=jnp.float32)
            continue
        if c_in == 128:
            parts = []
            for kw in range(4):
                j = (kw + 1) // 2
                lo = c_in if kw % 2 == 0 else 0
                parts.append(x_ref[:, pl.ds(hq, tr), hp, pl.ds(j, w_out),
                                   lo:lo + c_in])
        else:
            parts = [x_ref[:, pl.ds(hq, tr), hp, pl.ds(j, w_out), :]
                     for j in range(3)]
        patch = jnp.concatenate(parts, axis=-1).reshape(m, kk)
        acc = acc + jnp.dot(patch, w_ref[kh],
                            preferred_element_type=jnp.float32)
    _lrelu_store(acc, b_ref, o_ref, neg_slope)


def _pick_tiles(n, h_out, w_out, c_t):
    max_rows = max(1, (1024 * 1024) // (w_out * c_t * 4))
    tr = 1
    for d in range(1, h_out + 1):
        if h_out % d == 0 and d <= max_rows:
            tr = d
    nb = 1
    if tr == h_out:
        while (nb * 2 <= n and n % (nb * 2) == 0
               and 2 * nb * h_out * w_out <= 512):
            nb *= 2
    return tr, nb


def _conv_s1(x, w, b, *, ksize, padding, c_in, c_out, neg_slope=NEG_SLOPE):
    n, h, w2, _ = x.shape
    shared = w.ndim == 2
    h_out = h

    qp = jnp.pad(x, ((0, 0), (padding, padding), (1, 1), (0, 0)))
    tr, nb = _pick_tiles(n, h_out, w2, 2 * c_out)
    n_rb = h_out // tr

    if shared:
        w_spec = pl.BlockSpec(w.shape, lambda ci, bi, mi: (0, 0))
    else:
        w_spec = pl.BlockSpec(w.shape, lambda ci, bi, mi: (0, 0, 0))

    kern = functools.partial(_conv_s1_kernel, ksize=ksize, tr=tr, w2=w2,
                             nb=nb, c_in=c_in, shared=shared,
                             neg_slope=neg_slope)
    out = pl.pallas_call(
        kern,
        out_shape=jax.ShapeDtypeStruct((n, h_out * w2, 2 * c_out), x.dtype),
        grid=(1, n // nb, n_rb),
        in_specs=[
            pl.BlockSpec((nb,) + qp.shape[1:], lambda ci, bi, mi: (bi, 0, 0, 0)),
            w_spec,
            pl.BlockSpec((1, b.shape[-1]), lambda ci, bi, mi: (0, 0)),
        ],
        out_specs=pl.BlockSpec((nb, tr * w2, 2 * c_out),
                               lambda ci, bi, mi: (bi, mi, 0)),
        compiler_params=pltpu.CompilerParams(
            dimension_semantics=("parallel", "parallel", "parallel"),
            vmem_limit_bytes=100 * 1024 * 1024),
    )(qp, w, b)
    return out.reshape(n, h_out, w2, 2 * c_out)


def _conv_s2(x, w, b, *, neg_slope=NEG_SLOPE):
    n, h, w2, c2 = x.shape
    c_in = c2 // 2
    c_out = w.shape[-1]
    h_out = h // 2
    w_out = w2

    xpp = jnp.pad(x, ((0, 0), (1, 3), (1, 1), (0, 0)))
    xpp = xpp.reshape(n, (h + 4) // 2, 2, w2 + 2, c2)
    c_t = min(c_out, 256)
    n_cb = c_out // c_t
    tr, nb = _pick_tiles(n, h_out, w_out, c_t)
    n_rb = h_out // tr

    kern = functools.partial(_conv_s2_kernel, tr=tr, w_out=w_out, nb=nb,
                             c_in=c_in, neg_slope=neg_slope)
    out = pl.pallas_call(
        kern,
        out_shape=jax.ShapeDtypeStruct((n, h_out * w_out, c_out), x.dtype),
        grid=(n_cb, n // nb, n_rb),
        in_specs=[
            pl.BlockSpec((nb,) + xpp.shape[1:],
                         lambda ci, bi, mi: (bi, 0, 0, 0, 0)),
            pl.BlockSpec((4, w.shape[1], c_t), lambda ci, bi, mi: (0, 0, ci)),
            pl.BlockSpec((1, c_t), lambda ci, bi, mi: (0, ci)),
        ],
        out_specs=pl.BlockSpec((nb, tr * w_out, c_t),
                               lambda ci, bi, mi: (bi, mi, ci)),
        compiler_params=pltpu.CompilerParams(
            dimension_semantics=("parallel", "parallel", "parallel"),
            vmem_limit_bytes=100 * 1024 * 1024),
    )(xpp, w, b)
    return out.reshape(n, h_out, w_out // 2, 2 * c_out)


def _head_kernel(x_ref, w1_ref, b1_ref, w2_ref, b2_ref, o_ref, *, neg_slope):
    h = jnp.dot(x_ref[...], w1_ref[...],
                preferred_element_type=jnp.float32) + b1_ref[...]
    h = jnp.where(h > 0, h, neg_slope * h).astype(jnp.bfloat16)
    y = jnp.dot(h, w2_ref[...], preferred_element_type=jnp.float32)
    o_ref[...] = y + b2_ref[...]


def _head(feat, w1, b1, w2, b2, *, neg_slope=NEG_SLOPE):
    n, f = feat.shape
    out = pl.pallas_call(
        functools.partial(_head_kernel, neg_slope=neg_slope),
        out_shape=jax.ShapeDtypeStruct((n, 1), jnp.float32),
        grid=(1,),
        in_specs=[
            pl.BlockSpec((n, f), lambda i: (0, 0)),
            pl.BlockSpec(w1.shape, lambda i: (0, 0)),
            pl.BlockSpec((1, w1.shape[1]), lambda i: (0, 0)),
            pl.BlockSpec(w2.shape, lambda i: (0, 0)),
            pl.BlockSpec((1, 1), lambda i: (0, 0)),
        ],
        out_specs=pl.BlockSpec((n, 1), lambda i: (0, 0)),
        compiler_params=pltpu.CompilerParams(
            dimension_semantics=("arbitrary",),
            vmem_limit_bytes=32 * 1024 * 1024),
    )(feat, w1, b1, w2, b2)
    return out.reshape(-1)


def kernel(inp, label, cw0, cb0, cw1, cb1, cw2, cb2, cw3, cb3, cw4, cb4,
           cw5, cb5, cw6, cb6, cw7, cb7, cw8, cb8, cw9, cb9, w1, b1, w2, b2):
    cws = [cw0, cw1, cw2, cw3, cw4, cw5, cw6, cw7, cw8, cw9]
    cbs = [cb0, cb1, cb2, cb3, cb4, cb5, cb6, cb7, cb8, cb9]

    x = jnp.concatenate([jnp.transpose(inp, (0, 2, 3, 1)),
                         jnp.transpose(label, (0, 2, 3, 1))], axis=-1)
    x = jnp.pad(x, ((0, 0), (0, 0), (0, 0), (0, 2))).astype(jnp.bfloat16)
    n, hh, ww, _ = x.shape
    x = x.reshape(n, hh, ww // 2, 16)

    for li, (c_in, c_out, k, s, p) in enumerate(_CFG):
        w, b = cws[li], cbs[li]
        if li == 0:
            c_in = 8
            w = jnp.pad(w, ((0, 0), (0, 0), (0, 2), (0, 0)))
        if s == 1:
            wr = w.reshape(k, k * c_in, c_out)
            if c_in < 128:
                ze = jnp.zeros((k, (2 - p) * c_in, c_out), w.dtype)
                zet = jnp.zeros((k, (6 - k - (2 - p)) * c_in, c_out), w.dtype)
                zo = jnp.zeros((k, (3 - p) * c_in, c_out), w.dtype)
                zot = jnp.zeros((k, (6 - k - (3 - p)) * c_in, c_out), w.dtype)
                we = jnp.concatenate([ze, wr, zet], axis=1)
                wo = jnp.concatenate([zo, wr, zot], axis=1)
                wk = jnp.concatenate([we, wo], axis=-1)
                wk = wk.reshape(k * 6 * c_in, 2 * c_out).astype(jnp.bfloat16)
                bb = jnp.concatenate([b, b]).reshape(1, 2 * c_out)
            else:
                wk = wr.astype(jnp.bfloat16)
                bb = b.reshape(1, c_out)
            x = _conv_s1(x, wk, bb, ksize=k, padding=p, c_in=c_in,
                         c_out=c_out)
        else:
            wk = w.reshape(k, k * c_in, c_out)
            if c_in < 128:
                wk = jnp.pad(wk, ((0, 0), (c_in, c_in), (0, 0)))
            wk = wk.astype(jnp.bfloat16)
            x = _conv_s2(x, wk, b.reshape(1, c_out))

    n = x.shape[0]
    feat = x.reshape(n, -1)
    w1p = (w1.reshape(512, 4, 4, 64).transpose(1, 2, 0, 3)
           .reshape(4 * 4 * 512, 64)).astype(jnp.bfloat16)
    return _head(feat, w1p, b1.reshape(1, -1), w2.astype(jnp.bfloat16),
                 b2.reshape(1, 1))


# speedup vs baseline: 1.4476x; 1.4347x over previous
import functools

import jax
import jax.numpy as jnp
from jax.experimental import pallas as pl
from jax.experimental.pallas import tpu as pltpu

NEG_SLOPE = 0.2

_CFG = [
    (6,   64,  5, 1, 2),
    (64,  64,  4, 2, 1),
    (64,  128, 3, 1, 1),
    (128, 128, 4, 2, 1),
    (128, 256, 3, 1, 1),
    (256, 256, 4, 2, 1),
    (256, 512, 3, 1, 1),
    (512, 512, 4, 2, 1),
    (512, 512, 3, 1, 1),
    (512, 512, 4, 2, 1),
]


def _lrelu_store(acc, b_ref, o_ref, neg_slope):
    acc = acc + b_ref[...]
    acc = jnp.where(acc > 0, acc, neg_slope * acc)
    o_ref[...] = acc.astype(o_ref.dtype).reshape(o_ref.shape)


def _conv_s1_kernel(q_ref, w_ref, b_ref, o_ref, *, ksize, tr, w2, nb, c_in,
                    shared, neg_slope):
    m2 = nb * tr * w2
    r0 = pl.program_id(2) * tr

    def act(a):
        return jnp.where(a > 0, a, neg_slope * a)

    if shared:
        parts = [q_ref[:, pl.ds(r0 + kh, tr), pl.ds(j, w2), :]
                 for kh in range(ksize) for j in range(3)]
        patch = jnp.concatenate(parts, axis=-1).reshape(m2, w_ref.shape[0])
        acc = jnp.dot(patch, w_ref[...], preferred_element_type=jnp.float32)
        out = act(acc + b_ref[...])
    else:
        halves = []
        for off in (0, 1):
            c_out = w_ref.shape[-1]
            a = jnp.zeros((m2, c_out), jnp.float32)
            for kh in range(ksize):
                pieces = []
                for t in range(ksize):
                    idx = t + off + 1
                    j, half = idx // 2, idx % 2
                    pieces.append((t, q_ref[:, pl.ds(r0 + kh, tr),
                                            pl.ds(j, w2),
                                            half * c_in:(half + 1) * c_in]))
                if c_in >= 256:
                    for t, piece in pieces:
                        a = a + jnp.dot(
                            piece.reshape(m2, c_in),
                            w_ref[kh, t * c_in:(t + 1) * c_in, :],
                            preferred_element_type=jnp.float32)
                else:
                    patch = jnp.concatenate(
                        [p for _, p in pieces],
                        axis=-1).reshape(m2, ksize * c_in)
                    a = a + jnp.dot(patch, w_ref[kh],
                                    preferred_element_type=jnp.float32)
            halves.append(act(a + b_ref[...]))
        out = jnp.concatenate(halves, axis=-1)
    o_ref[...] = out.astype(o_ref.dtype).reshape(o_ref.shape)


def _conv_s2_kernel(x_ref, w_ref, b_ref, o_ref, *, tr, w_out, nb, c_in,
                    neg_slope):
    m = nb * tr * w_out
    kk = w_ref.shape[1]
    r0 = pl.program_id(2) * tr

    acc = jnp.zeros((m, w_ref.shape[-1]), jnp.float32)
    for kh in range(4):
        hq, hp = r0 + kh // 2, kh % 2
        if c_in >= 256:
            for kw in range(4):
                j = (kw + 1) // 2
                lo = c_in if kw % 2 == 0 else 0
                piece = x_ref[:, pl.ds(hq, tr), hp, pl.ds(j, w_out),
                              lo:lo + c_in]
                acc = acc + jnp.dot(piece.reshape(m, c_in),
                                    w_ref[kh, kw * c_in:(kw + 1) * c_in, :],
                                    preferred_element_type=jnp.float32)
            continue
        if c_in == 128:
            parts = []
            for kw in range(4):
                j = (kw + 1) // 2
                lo = c_in if kw % 2 == 0 else 0
                parts.append(x_ref[:, pl.ds(hq, tr), hp, pl.ds(j, w_out),
                                   lo:lo + c_in])
        else:
            parts = [x_ref[:, pl.ds(hq, tr), hp, pl.ds(j, w_out), :]
                     for j in range(3)]
        patch = jnp.concatenate(parts, axis=-1).reshape(m, kk)
        acc = acc + jnp.dot(patch, w_ref[kh],
                            preferred_element_type=jnp.float32)
    _lrelu_store(acc, b_ref, o_ref, neg_slope)


def _pick_tiles(n, h_out, w_out, c_t):
    max_rows = max(1, (1024 * 1024) // (w_out * c_t * 4))
    tr = 1
    for d in range(1, h_out + 1):
        if h_out % d == 0 and d <= max_rows:
            tr = d
    nb = 1
    if tr == h_out:
        while (nb * 2 <= n and n % (nb * 2) == 0
               and 2 * nb * h_out * w_out <= 512):
            nb *= 2
    return tr, nb


def _conv_s1(x, w, b, *, ksize, padding, c_in, c_out, neg_slope=NEG_SLOPE):
    n, h, w2, _ = x.shape
    shared = w.ndim == 2
    h_out = h

    qp = jnp.pad(x, ((0, 0), (padding, padding), (1, 1), (0, 0)))
    tr, nb = _pick_tiles(n, h_out, w2, 2 * c_out)
    n_rb = h_out // tr

    if shared:
        w_spec = pl.BlockSpec(w.shape, lambda ci, bi, mi: (0, 0))
    else:
        w_spec = pl.BlockSpec(w.shape, lambda ci, bi, mi: (0, 0, 0))

    kern = functools.partial(_conv_s1_kernel, ksize=ksize, tr=tr, w2=w2,
                             nb=nb, c_in=c_in, shared=shared,
                             neg_slope=neg_slope)
    out = pl.pallas_call(
        kern,
        out_shape=jax.ShapeDtypeStruct((n, h_out * w2, 2 * c_out), x.dtype),
        grid=(1, n // nb, n_rb),
        in_specs=[
            pl.BlockSpec((nb,) + qp.shape[1:], lambda ci, bi, mi: (bi, 0, 0, 0)),
            w_spec,
            pl.BlockSpec((1, b.shape[-1]), lambda ci, bi, mi: (0, 0)),
        ],
        out_specs=pl.BlockSpec((nb, tr * w2, 2 * c_out),
                               lambda ci, bi, mi: (bi, mi, 0)),
        compiler_params=pltpu.CompilerParams(
            dimension_semantics=("parallel", "parallel", "parallel"),
            vmem_limit_bytes=100 * 1024 * 1024),
    )(qp, w, b)
    return out.reshape(n, h_out, w2, 2 * c_out)


def _conv_s2(x, w, b, *, neg_slope=NEG_SLOPE):
    n, h, w2, c2 = x.shape
    c_in = c2 // 2
    c_out = w.shape[-1]
    h_out = h // 2
    w_out = w2

    xpp = jnp.pad(x, ((0, 0), (1, 3), (1, 1), (0, 0)))
    xpp = xpp.reshape(n, (h + 4) // 2, 2, w2 + 2, c2)
    c_t = min(c_out, 256)
    n_cb = c_out // c_t
    tr, nb = _pick_tiles(n, h_out, w_out, c_t)
    n_rb = h_out // tr

    kern = functools.partial(_conv_s2_kernel, tr=tr, w_out=w_out, nb=nb,
                             c_in=c_in, neg_slope=neg_slope)
    out = pl.pallas_call(
        kern,
        out_shape=jax.ShapeDtypeStruct((n, h_out * w_out, c_out), x.dtype),
        grid=(n_cb, n // nb, n_rb),
        in_specs=[
            pl.BlockSpec((nb,) + xpp.shape[1:],
                         lambda ci, bi, mi: (bi, 0, 0, 0, 0)),
            pl.BlockSpec((4, w.shape[1], c_t), lambda ci, bi, mi: (0, 0, ci)),
            pl.BlockSpec((1, c_t), lambda ci, bi, mi: (0, ci)),
        ],
        out_specs=pl.BlockSpec((nb, tr * w_out, c_t),
                               lambda ci, bi, mi: (bi, mi, ci)),
        compiler_params=pltpu.CompilerParams(
            dimension_semantics=("parallel", "parallel", "parallel"),
            vmem_limit_bytes=100 * 1024 * 1024),
    )(xpp, w, b)
    return out.reshape(n, h_out, w_out // 2, 2 * c_out)


def _head_kernel(x_ref, w1_ref, b1_ref, w2_ref, b2_ref, o_ref, *, neg_slope):
    h = jnp.dot(x_ref[...], w1_ref[...],
                preferred_element_type=jnp.float32) + b1_ref[...]
    h = jnp.where(h > 0, h, neg_slope * h).astype(jnp.bfloat16)
    y = jnp.dot(h, w2_ref[...], preferred_element_type=jnp.float32)
    o_ref[...] = y + b2_ref[...]


def _head(feat, w1, b1, w2, b2, *, neg_slope=NEG_SLOPE):
    n, f = feat.shape
    out = pl.pallas_call(
        functools.partial(_head_kernel, neg_slope=neg_slope),
        out_shape=jax.ShapeDtypeStruct((n, 1), jnp.float32),
        grid=(1,),
        in_specs=[
            pl.BlockSpec((n, f), lambda i: (0, 0)),
            pl.BlockSpec(w1.shape, lambda i: (0, 0)),
            pl.BlockSpec((1, w1.shape[1]), lambda i: (0, 0)),
            pl.BlockSpec(w2.shape, lambda i: (0, 0)),
            pl.BlockSpec((1, 1), lambda i: (0, 0)),
        ],
        out_specs=pl.BlockSpec((n, 1), lambda i: (0, 0)),
        compiler_params=pltpu.CompilerParams(
            dimension_semantics=("arbitrary",),
            vmem_limit_bytes=32 * 1024 * 1024),
    )(feat, w1, b1, w2, b2)
    return out.reshape(-1)


def kernel(inp, label, cw0, cb0, cw1, cb1, cw2, cb2, cw3, cb3, cw4, cb4,
           cw5, cb5, cw6, cb6, cw7, cb7, cw8, cb8, cw9, cb9, w1, b1, w2, b2):
    cws = [cw0, cw1, cw2, cw3, cw4, cw5, cw6, cw7, cw8, cw9]
    cbs = [cb0, cb1, cb2, cb3, cb4, cb5, cb6, cb7, cb8, cb9]

    x = jnp.concatenate([jnp.transpose(inp, (0, 2, 3, 1)),
                         jnp.transpose(label, (0, 2, 3, 1))], axis=-1)
    x = jnp.pad(x, ((0, 0), (0, 0), (0, 0), (0, 2))).astype(jnp.bfloat16)
    n, hh, ww, _ = x.shape
    x = x.reshape(n, hh, ww // 2, 16)

    import os as _os
    _stop = int(_os.environ.get("SCBAND_STOP_LAYER", "99"))
    for li, (c_in, c_out, k, s, p) in enumerate(_CFG):
        if li >= _stop:
            return x.astype(jnp.float32).sum(axis=(1, 2, 3))
        w, b = cws[li], cbs[li]
        if li == 0:
            c_in = 8
            w = jnp.pad(w, ((0, 0), (0, 0), (0, 2), (0, 0)))
        if s == 1:
            wr = w.reshape(k, k * c_in, c_out)
            if c_in < 128:
                ze = jnp.zeros((k, (2 - p) * c_in, c_out), w.dtype)
                zet = jnp.zeros((k, (6 - k - (2 - p)) * c_in, c_out), w.dtype)
                zo = jnp.zeros((k, (3 - p) * c_in, c_out), w.dtype)
                zot = jnp.zeros((k, (6 - k - (3 - p)) * c_in, c_out), w.dtype)
                we = jnp.concatenate([ze, wr, zet], axis=1)
                wo = jnp.concatenate([zo, wr, zot], axis=1)
                wk = jnp.concatenate([we, wo], axis=-1)
                wk = wk.reshape(k * 6 * c_in, 2 * c_out).astype(jnp.bfloat16)
                bb = jnp.concatenate([b, b]).reshape(1, 2 * c_out)
            else:
                wk = wr.astype(jnp.bfloat16)
                bb = b.reshape(1, c_out)
            x = _conv_s1(x, wk, bb, ksize=k, padding=p, c_in=c_in,
                         c_out=c_out)
        else:
            wk = w.reshape(k, k * c_in, c_out)
            if c_in < 128:
                wk = jnp.pad(wk, ((0, 0), (c_in, c_in), (0, 0)))
            wk = wk.astype(jnp.bfloat16)
            x = _conv_s2(x, wk, b.reshape(1, c_out))

    n = x.shape[0]
    feat = x.reshape(n, -1)
    w1p = (w1.reshape(512, 4, 4, 64).transpose(1, 2, 0, 3)
           .reshape(4 * 4 * 512, 64)).astype(jnp.bfloat16)
    return _head(feat, w1p, b1.reshape(1, -1), w2.astype(jnp.bfloat16),
                 b2.reshape(1, 1))
